# baseline (speedup 1.0000x reference)
import functools

import jax
import jax.numpy as jnp
from jax import lax
from jax.experimental import pallas as pl
from jax.experimental.pallas import tpu as pltpu
from jax.experimental.pallas import tpu_sc as plsc

N = 10000
E = 320000
D_IN = 128
D_HID = 16
N_CLASSES = 10

NC = 2
NS = 16
NW = NC * NS
L = 16

EPW = E // NW
SE = 1024
SLABS = [SE] * 9 + [EPW - 9 * SE]
SOFF = [i * SE for i in range(len(SLABS))]
NSLAB = len(SLABS)
NBUF = 3

NPR = 640
NP = NPR * L
RPT = NP // NS
CPT = NPR // NS

_MESH = plsc.VectorSubcoreMesh(
    core_axis_name="c", subcore_axis_name="s", num_cores=NC, num_subcores=NS
)
_SC_PARAMS = pltpu.CompilerParams(
    use_tc_tiling_on_sc=False, needs_layout_passes=False
)


def _zero_rows(buf, nrows):
    zero = jnp.zeros((L,), jnp.float32)

    def zb(i, c):
        buf[i, :] = zero
        return c

    lax.fori_loop(0, nrows, zb, 0)


def _wid():
    return lax.axis_index("s") * NC + lax.axis_index("c")


@functools.partial(
    pl.kernel,
    out_type=jax.ShapeDtypeStruct((NC, NPR, L), jnp.float32),
    mesh=_MESH,
    compiler_params=_SC_PARAMS,
    scratch_types=[
        pltpu.VMEM((NPR, L), jnp.float32),
        pltpu.VMEM((EPW,), jnp.int32),
        pltpu.VMEM((NPR,), jnp.int32),
        pltpu.SemaphoreType.DMA,
        pltpu.VMEM_SHARED((NPR, L), jnp.float32),
    ],
)
def _count_pass(ei, part, cnt, dbuf, idbuf, sem, acc):
    cid = lax.axis_index("c")
    sid = lax.axis_index("s")
    wid = _wid()

    pltpu.async_copy(ei.at[1, pl.ds(wid * EPW, EPW)], dbuf, sem)

    _zero_rows(cnt, NPR)
    base = lax.broadcasted_iota(jnp.int32, (L,), 0)

    def ib(i, c):
        idbuf[pl.ds(i * L, L)] = base + i * L
        return c

    lax.fori_loop(0, NPR // L, ib, 0)

    pltpu.sync_copy(cnt.at[pl.ds(0, CPT)], acc.at[pl.ds(sid * CPT, CPT)])

    pltpu.make_async_copy(ei.at[1, pl.ds(wid * EPW, EPW)], dbuf, sem).wait()
    plsc.subcore_barrier()

    ones = jnp.ones((L,), jnp.float32)

    def hist(i, c):
        v = dbuf[pl.ds(i * L, L)]
        lane = lax.shift_right_logical(lax.shift_right_logical(v, 7) * 205, 10)
        row = v - lane * NPR
        plsc.addupdate_scatter(cnt, [row, lane], ones)
        return c

    lax.fori_loop(0, EPW // L, hist, 0)

    pltpu.sync_copy(cnt, acc.at[idbuf], add=True)
    plsc.subcore_barrier()

    pltpu.sync_copy(acc.at[pl.ds(sid * CPT, CPT)], part.at[cid, pl.ds(sid * CPT, CPT)])


def _edge_ring(gtab, acc, sbuf, dbuf, rows, sems):
    def rbuf(b, n):
        return rows.at[pl.ds(b * SE, n)]

    def sidx(j):
        return sbuf.at[pl.ds(SOFF[j], SLABS[j])]

    def didx(j):
        return dbuf.at[pl.ds(SOFF[j], SLABS[j])]

    def fire_gather(j, b):
        pltpu.async_copy(gtab.at[sidx(j)], rbuf(b, SLABS[j]), sems[b])

    def wait_gather(j, b):
        pltpu.make_async_copy(gtab.at[sidx(j)], rbuf(b, SLABS[j]), sems[b]).wait()

    def fire_scatter(j, b):
        pltpu.async_copy(rbuf(b, SLABS[j]), acc.at[didx(j)], sems[b], add=True)

    def wait_scatter(j, b):
        pltpu.make_async_copy(rbuf(b, SLABS[j]), acc.at[didx(j)], sems[b]).wait()

    for b in range(NBUF):
        fire_gather(b, b)
    for j in range(NSLAB):
        b = j % NBUF
        wait_gather(j, b)
        fire_scatter(j, b)
        nj = j + NBUF
        if nj < NSLAB:
            wait_scatter(j, b)
            fire_gather(nj, b)
    for j in range(NSLAB - NBUF, NSLAB):
        wait_scatter(j, j % NBUF)


@functools.partial(
    pl.kernel,
    out_type=jax.ShapeDtypeStruct((NC, NP, L), jnp.float32),
    mesh=_MESH,
    compiler_params=_SC_PARAMS,
    scratch_types=[
        pltpu.VMEM((RPT, L), jnp.float32),
        pltpu.VMEM((EPW,), jnp.int32),
        pltpu.VMEM((EPW,), jnp.int32),
        pltpu.VMEM((NBUF * SE, L), jnp.float32),
        pltpu.SemaphoreType.DMA,
        pltpu.SemaphoreType.DMA,
        pltpu.SemaphoreType.DMA,
        pltpu.SemaphoreType.DMA,
        pltpu.VMEM_SHARED((NP, L), jnp.float32),
        pltpu.VMEM_SHARED((NP, L), jnp.float32),
    ],
)
def _message_pass(table, ei, part, zbuf, sbuf, dbuf, rows,
                  sem0, sem1, sem2, isem, acc, gtab):
    cid = lax.axis_index("c")
    sid = lax.axis_index("s")
    wid = _wid()

    pltpu.async_copy(
        table.at[pl.ds(sid * RPT, RPT)], gtab.at[pl.ds(sid * RPT, RPT)], isem
    )
    pltpu.async_copy(ei.at[0, pl.ds(wid * EPW, EPW)], sbuf, isem)
    pltpu.async_copy(ei.at[1, pl.ds(wid * EPW, EPW)], dbuf, isem)
    _zero_rows(zbuf, RPT)
    pltpu.sync_copy(zbuf, acc.at[pl.ds(sid * RPT, RPT)])
    pltpu.make_async_copy(
        table.at[pl.ds(sid * RPT, RPT)], gtab.at[pl.ds(sid * RPT, RPT)], isem
    ).wait()
    pltpu.make_async_copy(ei.at[0, pl.ds(wid * EPW, EPW)], sbuf, isem).wait()
    pltpu.make_async_copy(ei.at[1, pl.ds(wid * EPW, EPW)], dbuf, isem).wait()
    plsc.subcore_barrier()

    _edge_ring(gtab, acc, sbuf, dbuf, rows, [sem0, sem1, sem2])
    plsc.subcore_barrier()

    pltpu.sync_copy(acc.at[pl.ds(sid * RPT, RPT)], part.at[cid, pl.ds(sid * RPT, RPT)])


@functools.partial(
    pl.kernel,
    out_type=jax.ShapeDtypeStruct((NC, NP, L), jnp.float32),
    mesh=_MESH,
    compiler_params=_SC_PARAMS,
    scratch_types=[
        pltpu.VMEM((RPT, L), jnp.float32),
        pltpu.VMEM((RPT, L), jnp.float32),
        pltpu.VMEM((RPT, L), jnp.float32),
        pltpu.VMEM((RPT, L), jnp.float32),
        pltpu.VMEM((L,), jnp.float32),
        pltpu.VMEM((EPW,), jnp.int32),
        pltpu.VMEM((EPW,), jnp.int32),
        pltpu.VMEM((NBUF * SE, L), jnp.float32),
        pltpu.SemaphoreType.DMA,
        pltpu.SemaphoreType.DMA,
        pltpu.SemaphoreType.DMA,
        pltpu.SemaphoreType.DMA,
        pltpu.VMEM_SHARED((NP, L), jnp.float32),
        pltpu.VMEM_SHARED((NP, L), jnp.float32),
    ],
)
def _message_pass2(part1, g1, dis16, b1r, ei, part, abuf, bbuf, cbuf, dbuf16,
                   b1v, sbuf, dbuf, rows, sem0, sem1, sem2, isem, acc, gtab):
    cid = lax.axis_index("c")
    sid = lax.axis_index("s")
    wid = _wid()
    sl = pl.ds(sid * RPT, RPT)

    pltpu.async_copy(part1.at[0, sl], abuf, isem)
    pltpu.async_copy(part1.at[1, sl], bbuf, isem)
    pltpu.async_copy(g1.at[sl], cbuf, isem)
    pltpu.async_copy(dis16.at[sl], dbuf16, isem)
    pltpu.async_copy(ei.at[0, pl.ds(wid * EPW, EPW)], sbuf, isem)
    pltpu.async_copy(ei.at[1, pl.ds(wid * EPW, EPW)], dbuf, isem)
    pltpu.sync_copy(b1r.at[0], b1v)
    pltpu.make_async_copy(part1.at[0, sl], abuf, isem).wait()
    pltpu.make_async_copy(part1.at[1, sl], bbuf, isem).wait()
    pltpu.make_async_copy(g1.at[sl], cbuf, isem).wait()
    pltpu.make_async_copy(dis16.at[sl], dbuf16, isem).wait()

    b1vec = b1v[...]
    zero = jnp.zeros((L,), jnp.float32)

    def relu_row(i, c):
        a = abuf[i, :] + bbuf[i, :] + cbuf[i, :]
        d = dbuf16[i, :]
        abuf[i, :] = jnp.maximum(a * d + b1vec, 0.0) * d
        bbuf[i, :] = zero
        return c

    lax.fori_loop(0, RPT, relu_row, 0)

    pltpu.sync_copy(abuf, gtab.at[sl])
    pltpu.sync_copy(bbuf, acc.at[sl])
    pltpu.make_async_copy(ei.at[0, pl.ds(wid * EPW, EPW)], sbuf, isem).wait()
    pltpu.make_async_copy(ei.at[1, pl.ds(wid * EPW, EPW)], dbuf, isem).wait()
    plsc.subcore_barrier()

    _edge_ring(gtab, acc, sbuf, dbuf, rows, [sem0, sem1, sem2])
    plsc.subcore_barrier()

    pltpu.sync_copy(acc.at[sl], part.at[cid, sl])


def _unpack_dis(cnt_ref):
    c = cnt_ref[0] + cnt_ref[1]
    disp = lax.rsqrt(c + 1.0)
    return jnp.concatenate(
        [jnp.broadcast_to(disp[:, l:l + 1], (NPR, L)) for l in range(L)], axis=0
    )


def _t1a_body(x_ref, w1_ref, h_ref):
    h_ref[...] = jnp.dot(
        x_ref[...], w1_ref[...], preferred_element_type=jnp.float32
    )


def _t1b_body(cnt_ref, h_ref, g1_ref, dis_ref):
    dis16 = _unpack_dis(cnt_ref)
    g1_ref[:N] = h_ref[...] * dis16[:N]
    g1_ref[N:] = jnp.zeros((NP - N, D_HID), jnp.float32)
    dis_ref[...] = dis16


def _t3_body(part2_ref, part1_ref, g1_ref, dis_ref, b1_ref, w2_ref, b2_ref,
             o_ref):
    d = dis_ref[:N]
    acc1 = part1_ref[0][:N] + part1_ref[1][:N] + g1_ref[:N]
    r = jnp.maximum(acc1 * d + b1_ref[...], 0.0) * d
    acc2 = part2_ref[0][:N] + part2_ref[1][:N] + r
    o_ref[...] = (
        jnp.dot(acc2 * d, w2_ref[...], preferred_element_type=jnp.float32)
        + b2_ref[...]
    )


_t1a = pl.pallas_call(
    _t1a_body,
    out_shape=jax.ShapeDtypeStruct((N, D_HID), jnp.float32),
)
_t1b = pl.pallas_call(
    _t1b_body,
    out_shape=[
        jax.ShapeDtypeStruct((NP, D_HID), jnp.float32),
        jax.ShapeDtypeStruct((NP, D_HID), jnp.float32),
    ],
)
_t3 = pl.pallas_call(
    _t3_body,
    out_shape=jax.ShapeDtypeStruct((N, N_CLASSES), jnp.float32),
)


def kernel(x, edge_index, W1, b1, W2, b2):
    ei = edge_index.astype(jnp.int32)
    b1r = b1.reshape(1, D_HID)

    h = _t1a(x, W1)
    part_cnt = _count_pass(ei)
    g1, dis16 = _t1b(part_cnt, h)
    part1 = _message_pass(g1, ei)
    part2 = _message_pass2(part1, g1, dis16, b1r, ei)
    out = _t3(part2, part1, g1, dis16, b1r, W2, b2.reshape(1, N_CLASSES))
    return out


# speedup vs baseline: 1.0579x; 1.0579x over previous
import functools

import jax
import jax.numpy as jnp
from jax import lax
from jax.experimental import pallas as pl
from jax.experimental.pallas import tpu as pltpu
from jax.experimental.pallas import tpu_sc as plsc

N = 10000
E = 320000
D_IN = 128
D_HID = 16
N_CLASSES = 10

NC = 2
NS = 16
NW = NC * NS
L = 16

EPW = E // NW
SE = 1024
SLABS = [SE] * 9 + [EPW - 9 * SE]
SOFF = [i * SE for i in range(len(SLABS))]
NSLAB = len(SLABS)
NBUF = 3

NPR = 640
NP = NPR * L
RPT = NP // NS
CPT = NPR // NS

_MESH = plsc.VectorSubcoreMesh(
    core_axis_name="c", subcore_axis_name="s", num_cores=NC, num_subcores=NS
)
_SC_PARAMS = pltpu.CompilerParams(
    use_tc_tiling_on_sc=False, needs_layout_passes=False
)


def _zero_rows(buf, nrows):
    zero = jnp.zeros((L,), jnp.float32)

    def zb(i, c):
        for k in range(4):
            buf[i * 4 + k, :] = zero
        return c

    lax.fori_loop(0, nrows // 4, zb, 0)


def _wid():
    return lax.axis_index("s") * NC + lax.axis_index("c")


@functools.partial(
    pl.kernel,
    out_type=jax.ShapeDtypeStruct((NC, NPR, L), jnp.float32),
    mesh=_MESH,
    compiler_params=_SC_PARAMS,
    scratch_types=[
        pltpu.VMEM((NPR, L), jnp.float32),
        pltpu.VMEM((EPW,), jnp.int32),
        pltpu.VMEM((NPR,), jnp.int32),
        pltpu.SemaphoreType.DMA,
        pltpu.VMEM_SHARED((NPR, L), jnp.float32),
    ],
)
def _count_pass(ei, part, cnt, dbuf, idbuf, sem, acc):
    cid = lax.axis_index("c")
    sid = lax.axis_index("s")
    wid = _wid()

    pltpu.async_copy(ei.at[1, pl.ds(wid * EPW, EPW)], dbuf, sem)

    _zero_rows(cnt, NPR)
    base = lax.broadcasted_iota(jnp.int32, (L,), 0)

    def ib(i, c):
        idbuf[pl.ds(i * L, L)] = base + i * L
        return c

    lax.fori_loop(0, NPR // L, ib, 0)

    pltpu.sync_copy(cnt.at[pl.ds(0, CPT)], acc.at[pl.ds(sid * CPT, CPT)])

    pltpu.make_async_copy(ei.at[1, pl.ds(wid * EPW, EPW)], dbuf, sem).wait()
    plsc.subcore_barrier()

    ones = jnp.ones((L,), jnp.float32)

    def hist(i, c):
        for k in range(4):
            v = dbuf[pl.ds((i * 4 + k) * L, L)]
            lane = lax.shift_right_logical(
                lax.shift_right_logical(v, 7) * 205, 10
            )
            row = v - lane * NPR
            plsc.addupdate_scatter(cnt, [row, lane], ones)
        return c

    lax.fori_loop(0, EPW // L // 4, hist, 0)

    pltpu.sync_copy(cnt, acc.at[idbuf], add=True)
    plsc.subcore_barrier()

    pltpu.sync_copy(acc.at[pl.ds(sid * CPT, CPT)], part.at[cid, pl.ds(sid * CPT, CPT)])


def _edge_ring(gtab, acc, sbuf, dbuf, rows, sems):
    def rbuf(b, n):
        return rows.at[pl.ds(b * SE, n)]

    def sidx(j):
        return sbuf.at[pl.ds(SOFF[j], SLABS[j])]

    def didx(j):
        return dbuf.at[pl.ds(SOFF[j], SLABS[j])]

    def fire_gather(j, b):
        pltpu.async_copy(gtab.at[sidx(j)], rbuf(b, SLABS[j]), sems[b])

    def wait_gather(j, b):
        pltpu.make_async_copy(gtab.at[sidx(j)], rbuf(b, SLABS[j]), sems[b]).wait()

    def fire_scatter(j, b):
        pltpu.async_copy(rbuf(b, SLABS[j]), acc.at[didx(j)], sems[b], add=True)

    def wait_scatter(j, b):
        pltpu.make_async_copy(rbuf(b, SLABS[j]), acc.at[didx(j)], sems[b]).wait()

    for b in range(NBUF):
        fire_gather(b, b)
    for j in range(NSLAB):
        b = j % NBUF
        wait_gather(j, b)
        fire_scatter(j, b)
        nj = j + NBUF
        if nj < NSLAB:
            wait_scatter(j, b)
            fire_gather(nj, b)
    for j in range(NSLAB - NBUF, NSLAB):
        wait_scatter(j, j % NBUF)


@functools.partial(
    pl.kernel,
    out_type=jax.ShapeDtypeStruct((NC, NP, L), jnp.float32),
    mesh=_MESH,
    compiler_params=_SC_PARAMS,
    scratch_types=[
        pltpu.VMEM((RPT, L), jnp.float32),
        pltpu.VMEM((EPW,), jnp.int32),
        pltpu.VMEM((EPW,), jnp.int32),
        pltpu.VMEM((NBUF * SE, L), jnp.float32),
        pltpu.SemaphoreType.DMA,
        pltpu.SemaphoreType.DMA,
        pltpu.SemaphoreType.DMA,
        pltpu.SemaphoreType.DMA,
        pltpu.VMEM_SHARED((NP, L), jnp.float32),
        pltpu.VMEM_SHARED((NP, L), jnp.float32),
    ],
)
def _message_pass(table, ei, part, zbuf, sbuf, dbuf, rows,
                  sem0, sem1, sem2, isem, acc, gtab):
    cid = lax.axis_index("c")
    sid = lax.axis_index("s")
    wid = _wid()

    pltpu.async_copy(
        table.at[pl.ds(sid * RPT, RPT)], gtab.at[pl.ds(sid * RPT, RPT)], isem
    )
    pltpu.async_copy(ei.at[0, pl.ds(wid * EPW, EPW)], sbuf, isem)
    pltpu.async_copy(ei.at[1, pl.ds(wid * EPW, EPW)], dbuf, isem)
    _zero_rows(zbuf, RPT)
    pltpu.sync_copy(zbuf, acc.at[pl.ds(sid * RPT, RPT)])
    pltpu.make_async_copy(
        table.at[pl.ds(sid * RPT, RPT)], gtab.at[pl.ds(sid * RPT, RPT)], isem
    ).wait()
    pltpu.make_async_copy(ei.at[0, pl.ds(wid * EPW, EPW)], sbuf, isem).wait()
    pltpu.make_async_copy(ei.at[1, pl.ds(wid * EPW, EPW)], dbuf, isem).wait()
    plsc.subcore_barrier()

    _edge_ring(gtab, acc, sbuf, dbuf, rows, [sem0, sem1, sem2])
    plsc.subcore_barrier()

    pltpu.sync_copy(acc.at[pl.ds(sid * RPT, RPT)], part.at[cid, pl.ds(sid * RPT, RPT)])


@functools.partial(
    pl.kernel,
    out_type=jax.ShapeDtypeStruct((NC, NP, L), jnp.float32),
    mesh=_MESH,
    compiler_params=_SC_PARAMS,
    scratch_types=[
        pltpu.VMEM((RPT, L), jnp.float32),
        pltpu.VMEM((RPT, L), jnp.float32),
        pltpu.VMEM((RPT, L), jnp.float32),
        pltpu.VMEM((RPT, L), jnp.float32),
        pltpu.VMEM((L,), jnp.float32),
        pltpu.VMEM((EPW,), jnp.int32),
        pltpu.VMEM((EPW,), jnp.int32),
        pltpu.VMEM((NBUF * SE, L), jnp.float32),
        pltpu.SemaphoreType.DMA,
        pltpu.SemaphoreType.DMA,
        pltpu.SemaphoreType.DMA,
        pltpu.SemaphoreType.DMA,
        pltpu.VMEM_SHARED((NP, L), jnp.float32),
        pltpu.VMEM_SHARED((NP, L), jnp.float32),
    ],
)
def _message_pass2(part1, g1, dis16, b1r, ei, part, abuf, bbuf, cbuf, dbuf16,
                   b1v, sbuf, dbuf, rows, sem0, sem1, sem2, isem, acc, gtab):
    cid = lax.axis_index("c")
    sid = lax.axis_index("s")
    wid = _wid()
    sl = pl.ds(sid * RPT, RPT)

    pltpu.async_copy(part1.at[0, sl], abuf, isem)
    pltpu.async_copy(part1.at[1, sl], bbuf, isem)
    pltpu.async_copy(g1.at[sl], cbuf, isem)
    pltpu.async_copy(dis16.at[sl], dbuf16, isem)
    pltpu.async_copy(ei.at[0, pl.ds(wid * EPW, EPW)], sbuf, isem)
    pltpu.async_copy(ei.at[1, pl.ds(wid * EPW, EPW)], dbuf, isem)
    pltpu.sync_copy(b1r.at[0], b1v)
    pltpu.make_async_copy(part1.at[0, sl], abuf, isem).wait()
    pltpu.make_async_copy(part1.at[1, sl], bbuf, isem).wait()
    pltpu.make_async_copy(g1.at[sl], cbuf, isem).wait()
    pltpu.make_async_copy(dis16.at[sl], dbuf16, isem).wait()

    b1vec = b1v[...]
    zero = jnp.zeros((L,), jnp.float32)

    def relu_row(i, c):
        for k in range(4):
            j = i * 4 + k
            a = abuf[j, :] + bbuf[j, :] + cbuf[j, :]
            d = dbuf16[j, :]
            abuf[j, :] = jnp.maximum(a * d + b1vec, 0.0) * d
            bbuf[j, :] = zero
        return c

    lax.fori_loop(0, RPT // 4, relu_row, 0)

    pltpu.sync_copy(abuf, gtab.at[sl])
    pltpu.sync_copy(bbuf, acc.at[sl])
    pltpu.make_async_copy(ei.at[0, pl.ds(wid * EPW, EPW)], sbuf, isem).wait()
    pltpu.make_async_copy(ei.at[1, pl.ds(wid * EPW, EPW)], dbuf, isem).wait()
    plsc.subcore_barrier()

    _edge_ring(gtab, acc, sbuf, dbuf, rows, [sem0, sem1, sem2])
    plsc.subcore_barrier()

    pltpu.sync_copy(acc.at[sl], part.at[cid, sl])


def _unpack_dis(cnt_ref):
    c = cnt_ref[0] + cnt_ref[1]
    disp = lax.rsqrt(c + 1.0)
    return jnp.concatenate(
        [jnp.broadcast_to(disp[:, l:l + 1], (NPR, L)) for l in range(L)], axis=0
    )


def _t1a_body(x_ref, w1_ref, h_ref):
    h_ref[...] = jnp.dot(
        x_ref[...], w1_ref[...], preferred_element_type=jnp.float32
    )


def _t1b_body(cnt_ref, h_ref, g1_ref, dis_ref):
    dis16 = _unpack_dis(cnt_ref)
    g1_ref[:N] = h_ref[...] * dis16[:N]
    g1_ref[N:] = jnp.zeros((NP - N, D_HID), jnp.float32)
    dis_ref[...] = dis16


def _t3_body(part2_ref, part1_ref, g1_ref, dis_ref, b1_ref, w2_ref, b2_ref,
             o_ref):
    d = dis_ref[:N]
    acc1 = part1_ref[0][:N] + part1_ref[1][:N] + g1_ref[:N]
    r = jnp.maximum(acc1 * d + b1_ref[...], 0.0) * d
    acc2 = part2_ref[0][:N] + part2_ref[1][:N] + r
    o_ref[...] = (
        jnp.dot(acc2 * d, w2_ref[...], preferred_element_type=jnp.float32)
        + b2_ref[...]
    )


_t1a = pl.pallas_call(
    _t1a_body,
    out_shape=jax.ShapeDtypeStruct((N, D_HID), jnp.float32),
)
_t1b = pl.pallas_call(
    _t1b_body,
    out_shape=[
        jax.ShapeDtypeStruct((NP, D_HID), jnp.float32),
        jax.ShapeDtypeStruct((NP, D_HID), jnp.float32),
    ],
)
_t3 = pl.pallas_call(
    _t3_body,
    out_shape=jax.ShapeDtypeStruct((N, N_CLASSES), jnp.float32),
)


def kernel(x, edge_index, W1, b1, W2, b2):
    ei = edge_index.astype(jnp.int32)
    b1r = b1.reshape(1, D_HID)

    h = _t1a(x, W1)
    part_cnt = _count_pass(ei)
    g1, dis16 = _t1b(part_cnt, h)
    part1 = _message_pass(g1, ei)
    part2 = _message_pass2(part1, g1, dis16, b1r, ei)
    out = _t3(part2, part1, g1, dis16, b1r, W2, b2.reshape(1, N_CLASSES))
    return out


# speedup vs baseline: 1.0611x; 1.0030x over previous
import functools

import jax
import jax.numpy as jnp
from jax import lax
from jax.experimental import pallas as pl
from jax.experimental.pallas import tpu as pltpu
from jax.experimental.pallas import tpu_sc as plsc

N = 10000
E = 320000
D_IN = 128
D_HID = 16
N_CLASSES = 10

NC = 2
NS = 16
NW = NC * NS
L = 16

EPW = E // NW
SE = 1024
SLABS = [SE] * 9 + [EPW - 9 * SE]
SOFF = [i * SE for i in range(len(SLABS))]
NSLAB = len(SLABS)
NBUF = 3

NPR = 640
NP = NPR * L
RPT = NP // NS
CPT = NPR // NS

_MESH = plsc.VectorSubcoreMesh(
    core_axis_name="c", subcore_axis_name="s", num_cores=NC, num_subcores=NS
)
_SC_PARAMS = pltpu.CompilerParams(
    use_tc_tiling_on_sc=False, needs_layout_passes=False
)


def _zero_rows(buf, nrows):
    zero = jnp.zeros((L,), jnp.float32)

    def zb(i, c):
        for k in range(4):
            buf[i * 4 + k, :] = zero
        return c

    lax.fori_loop(0, nrows // 4, zb, 0)


def _wid():
    return lax.axis_index("s") * NC + lax.axis_index("c")


@functools.partial(
    pl.kernel,
    out_type=jax.ShapeDtypeStruct((NC, NPR, L), jnp.float32),
    mesh=_MESH,
    compiler_params=_SC_PARAMS,
    scratch_types=[
        pltpu.VMEM((NPR, L), jnp.float32),
        pltpu.VMEM((EPW,), jnp.int32),
        pltpu.VMEM((NPR,), jnp.int32),
        pltpu.SemaphoreType.DMA,
        pltpu.VMEM_SHARED((NPR, L), jnp.float32),
    ],
)
def _count_pass(ei, part, cnt, dbuf, idbuf, sem, acc):
    cid = lax.axis_index("c")
    sid = lax.axis_index("s")
    wid = _wid()

    pltpu.async_copy(ei.at[1, pl.ds(wid * EPW, EPW)], dbuf, sem)

    _zero_rows(cnt, NPR)
    base = lax.broadcasted_iota(jnp.int32, (L,), 0)

    def ib(i, c):
        idbuf[pl.ds(i * L, L)] = base + i * L
        return c

    lax.fori_loop(0, NPR // L, ib, 0)

    pltpu.sync_copy(cnt.at[pl.ds(0, CPT)], acc.at[pl.ds(sid * CPT, CPT)])

    pltpu.make_async_copy(ei.at[1, pl.ds(wid * EPW, EPW)], dbuf, sem).wait()
    plsc.subcore_barrier()

    ones = jnp.ones((L,), jnp.float32)

    def hist1(i):
        v = dbuf[pl.ds(i * L, L)]
        lane = lax.shift_right_logical(lax.shift_right_logical(v, 7) * 205, 10)
        row = v - lane * NPR
        plsc.addupdate_scatter(cnt, [row, lane], ones)

    def hist(i, c):
        for k in range(4):
            hist1(i * 4 + k)
        return c

    NV = EPW // L
    lax.fori_loop(0, NV // 4, hist, 0)
    for t in range((NV // 4) * 4, NV):
        hist1(t)

    pltpu.sync_copy(cnt, acc.at[idbuf], add=True)
    plsc.subcore_barrier()

    pltpu.sync_copy(acc.at[pl.ds(sid * CPT, CPT)], part.at[cid, pl.ds(sid * CPT, CPT)])


def _edge_ring(gtab, acc, sbuf, dbuf, rows, sems):
    def rbuf(b, n):
        return rows.at[pl.ds(b * SE, n)]

    def sidx(j):
        return sbuf.at[pl.ds(SOFF[j], SLABS[j])]

    def didx(j):
        return dbuf.at[pl.ds(SOFF[j], SLABS[j])]

    def fire_gather(j, b):
        pltpu.async_copy(gtab.at[sidx(j)], rbuf(b, SLABS[j]), sems[b])

    def wait_gather(j, b):
        pltpu.make_async_copy(gtab.at[sidx(j)], rbuf(b, SLABS[j]), sems[b]).wait()

    def fire_scatter(j, b):
        pltpu.async_copy(rbuf(b, SLABS[j]), acc.at[didx(j)], sems[b], add=True)

    def wait_scatter(j, b):
        pltpu.make_async_copy(rbuf(b, SLABS[j]), acc.at[didx(j)], sems[b]).wait()

    for b in range(NBUF):
        fire_gather(b, b)
    for j in range(NSLAB):
        b = j % NBUF
        wait_gather(j, b)
        fire_scatter(j, b)
        nj = j + NBUF
        if nj < NSLAB:
            wait_scatter(j, b)
            fire_gather(nj, b)
    for j in range(NSLAB - NBUF, NSLAB):
        wait_scatter(j, j % NBUF)


@functools.partial(
    pl.kernel,
    out_type=jax.ShapeDtypeStruct((NC, NP, L), jnp.float32),
    mesh=_MESH,
    compiler_params=_SC_PARAMS,
    scratch_types=[
        pltpu.VMEM((RPT, L), jnp.float32),
        pltpu.VMEM((EPW,), jnp.int32),
        pltpu.VMEM((EPW,), jnp.int32),
        pltpu.VMEM((NBUF * SE, L), jnp.float32),
        pltpu.SemaphoreType.DMA,
        pltpu.SemaphoreType.DMA,
        pltpu.SemaphoreType.DMA,
        pltpu.SemaphoreType.DMA,
        pltpu.VMEM_SHARED((NP, L), jnp.float32),
        pltpu.VMEM_SHARED((NP, L), jnp.float32),
    ],
)
def _message_pass(table, ei, part, zbuf, sbuf, dbuf, rows,
                  sem0, sem1, sem2, isem, acc, gtab):
    cid = lax.axis_index("c")
    sid = lax.axis_index("s")
    wid = _wid()

    pltpu.async_copy(
        table.at[pl.ds(sid * RPT, RPT)], gtab.at[pl.ds(sid * RPT, RPT)], isem
    )
    pltpu.async_copy(ei.at[0, pl.ds(wid * EPW, EPW)], sbuf, isem)
    pltpu.async_copy(ei.at[1, pl.ds(wid * EPW, EPW)], dbuf, isem)
    _zero_rows(zbuf, RPT)
    pltpu.sync_copy(zbuf, acc.at[pl.ds(sid * RPT, RPT)])
    pltpu.make_async_copy(
        table.at[pl.ds(sid * RPT, RPT)], gtab.at[pl.ds(sid * RPT, RPT)], isem
    ).wait()
    pltpu.make_async_copy(ei.at[0, pl.ds(wid * EPW, EPW)], sbuf, isem).wait()
    pltpu.make_async_copy(ei.at[1, pl.ds(wid * EPW, EPW)], dbuf, isem).wait()
    plsc.subcore_barrier()

    _edge_ring(gtab, acc, sbuf, dbuf, rows, [sem0, sem1, sem2])
    plsc.subcore_barrier()

    pltpu.sync_copy(acc.at[pl.ds(sid * RPT, RPT)], part.at[cid, pl.ds(sid * RPT, RPT)])


@functools.partial(
    pl.kernel,
    out_type=jax.ShapeDtypeStruct((NC, NP, L), jnp.float32),
    mesh=_MESH,
    compiler_params=_SC_PARAMS,
    scratch_types=[
        pltpu.VMEM((RPT, L), jnp.float32),
        pltpu.VMEM((RPT, L), jnp.float32),
        pltpu.VMEM((RPT, L), jnp.float32),
        pltpu.VMEM((RPT, L), jnp.float32),
        pltpu.VMEM((L,), jnp.float32),
        pltpu.VMEM((EPW,), jnp.int32),
        pltpu.VMEM((EPW,), jnp.int32),
        pltpu.VMEM((NBUF * SE, L), jnp.float32),
        pltpu.SemaphoreType.DMA,
        pltpu.SemaphoreType.DMA,
        pltpu.SemaphoreType.DMA,
        pltpu.SemaphoreType.DMA,
        pltpu.VMEM_SHARED((NP, L), jnp.float32),
        pltpu.VMEM_SHARED((NP, L), jnp.float32),
    ],
)
def _message_pass2(part1, g1, dis16, b1r, ei, part, abuf, bbuf, cbuf, dbuf16,
                   b1v, sbuf, dbuf, rows, sem0, sem1, sem2, isem, acc, gtab):
    cid = lax.axis_index("c")
    sid = lax.axis_index("s")
    wid = _wid()
    sl = pl.ds(sid * RPT, RPT)

    pltpu.async_copy(part1.at[0, sl], abuf, isem)
    pltpu.async_copy(part1.at[1, sl], bbuf, isem)
    pltpu.async_copy(g1.at[sl], cbuf, isem)
    pltpu.async_copy(dis16.at[sl], dbuf16, isem)
    pltpu.async_copy(ei.at[0, pl.ds(wid * EPW, EPW)], sbuf, isem)
    pltpu.async_copy(ei.at[1, pl.ds(wid * EPW, EPW)], dbuf, isem)
    pltpu.sync_copy(b1r.at[0], b1v)
    pltpu.make_async_copy(part1.at[0, sl], abuf, isem).wait()
    pltpu.make_async_copy(part1.at[1, sl], bbuf, isem).wait()
    pltpu.make_async_copy(g1.at[sl], cbuf, isem).wait()
    pltpu.make_async_copy(dis16.at[sl], dbuf16, isem).wait()

    b1vec = b1v[...]
    zero = jnp.zeros((L,), jnp.float32)

    def relu_row(i, c):
        for k in range(4):
            j = i * 4 + k
            a = abuf[j, :] + bbuf[j, :] + cbuf[j, :]
            d = dbuf16[j, :]
            abuf[j, :] = jnp.maximum(a * d + b1vec, 0.0) * d
            bbuf[j, :] = zero
        return c

    lax.fori_loop(0, RPT // 4, relu_row, 0)

    pltpu.sync_copy(abuf, gtab.at[sl])
    pltpu.sync_copy(bbuf, acc.at[sl])
    pltpu.make_async_copy(ei.at[0, pl.ds(wid * EPW, EPW)], sbuf, isem).wait()
    pltpu.make_async_copy(ei.at[1, pl.ds(wid * EPW, EPW)], dbuf, isem).wait()
    plsc.subcore_barrier()

    _edge_ring(gtab, acc, sbuf, dbuf, rows, [sem0, sem1, sem2])
    plsc.subcore_barrier()

    pltpu.sync_copy(acc.at[sl], part.at[cid, sl])


def _unpack_dis(cnt_ref):
    c = cnt_ref[0] + cnt_ref[1]
    disp = lax.rsqrt(c + 1.0)
    return jnp.concatenate(
        [jnp.broadcast_to(disp[:, l:l + 1], (NPR, L)) for l in range(L)], axis=0
    )


def _t1a_body(x_ref, w1_ref, h_ref):
    h_ref[...] = jnp.dot(
        x_ref[...], w1_ref[...], preferred_element_type=jnp.float32
    )


def _t1b_body(cnt_ref, h_ref, g1_ref, dis_ref):
    dis16 = _unpack_dis(cnt_ref)
    g1_ref[:N] = h_ref[...] * dis16[:N]
    g1_ref[N:] = jnp.zeros((NP - N, D_HID), jnp.float32)
    dis_ref[...] = dis16


def _t3_body(part2_ref, part1_ref, g1_ref, dis_ref, b1_ref, w2_ref, b2_ref,
             o_ref):
    d = dis_ref[:N]
    acc1 = part1_ref[0][:N] + part1_ref[1][:N] + g1_ref[:N]
    r = jnp.maximum(acc1 * d + b1_ref[...], 0.0) * d
    acc2 = part2_ref[0][:N] + part2_ref[1][:N] + r
    o_ref[...] = (
        jnp.dot(acc2 * d, w2_ref[...], preferred_element_type=jnp.float32)
        + b2_ref[...]
    )


_t1a = pl.pallas_call(
    _t1a_body,
    out_shape=jax.ShapeDtypeStruct((N, D_HID), jnp.float32),
)
_t1b = pl.pallas_call(
    _t1b_body,
    out_shape=[
        jax.ShapeDtypeStruct((NP, D_HID), jnp.float32),
        jax.ShapeDtypeStruct((NP, D_HID), jnp.float32),
    ],
)
_t3 = pl.pallas_call(
    _t3_body,
    out_shape=jax.ShapeDtypeStruct((N, N_CLASSES), jnp.float32),
)


def kernel(x, edge_index, W1, b1, W2, b2):
    ei = edge_index.astype(jnp.int32)
    b1r = b1.reshape(1, D_HID)

    h = _t1a(x, W1)
    part_cnt = _count_pass(ei)
    g1, dis16 = _t1b(part_cnt, h)
    part1 = _message_pass(g1, ei)
    part2 = _message_pass2(part1, g1, dis16, b1r, ei)
    out = _t3(part2, part1, g1, dis16, b1r, W2, b2.reshape(1, N_CLASSES))
    return out
